# baseline (speedup 1.0000x reference)
import functools

import jax
import jax.numpy as jnp
from jax import lax
from jax.experimental import pallas as pl
from jax.experimental.pallas import tpu as pltpu
from jax.experimental.pallas import tpu_sc as plsc

B = 4096
L = 200
V = 100000
D = 128

NC = 2
NS = 16
LANES = 16
NW = NC * NS
BPW = B // NW
CHUNKS = ((0, 104), (104, 96))
NBUF = 2
DW = D // 2
NVREG = D // 32

CAST_ROWS = 800


def _cast_body(x_ref, o_ref):
    bits = lax.bitcast_convert_type(x_ref[...], jnp.int32)
    r = bits + 0x7FFF + jnp.bitwise_and(
        lax.shift_right_logical(bits, 16), jnp.int32(1))
    r4 = r.reshape(CAST_ROWS, D // 32, 2, 16)
    lo = lax.shift_right_logical(r4[:, :, 0, :], 16)
    hi = jnp.bitwise_and(r4[:, :, 1, :], jnp.int32(-65536))
    o_ref[...] = jnp.bitwise_or(hi, lo).reshape(CAST_ROWS, DW)


def _pack_table(table):
    return pl.pallas_call(
        _cast_body,
        grid=(V // CAST_ROWS,),
        in_specs=[pl.BlockSpec((CAST_ROWS, D), lambda i: (i, 0))],
        out_specs=pl.BlockSpec((CAST_ROWS, DW), lambda i: (i, 0)),
        out_shape=jax.ShapeDtypeStruct((V, DW), jnp.int32),
    )(table)


def _bow_body(idx_hbm, table_hbm, out_hbm, idx_v, buf_v, out_v, sem0, sem1):
    wid = lax.axis_index("s") * NC + lax.axis_index("c")
    sems = (sem0, sem1)
    inv = jnp.full((LANES,), 1.0 / L, dtype=jnp.float32)

    pltpu.sync_copy(idx_hbm.at[pl.ds(wid * BPW, BPW)], idx_v)

    def start_gather(slot, bag):
        for off, ch in CHUNKS:
            pltpu.make_async_copy(
                table_hbm.at[idx_v.at[bag, pl.ds(off, ch)]],
                buf_v.at[slot, pl.ds(off, ch)],
                sems[slot],
            ).start()

    def drain(slot):
        for off, ch in CHUNKS:
            pltpu.make_async_copy(
                table_hbm.at[idx_v.at[0, pl.ds(off, ch)]],
                buf_v.at[slot, pl.ds(off, ch)],
                sems[slot],
            ).wait()

    def consume(slot, bag):
        UNROLL = 4

        def row_add(i, accs):
            l = i * UNROLL
            out = []
            for k in range(NVREG):
                lo, hi = accs[2 * k], accs[2 * k + 1]
                los, his = [], []
                for u in range(UNROLL):
                    x = buf_v[slot, l + u, pl.ds(k * LANES, LANES)]
                    a, b = plsc.unpack(
                        plsc.bitcast(x, jnp.bfloat16),
                        format=plsc.PackFormat.INTERLEAVED)
                    los.append(a)
                    his.append(b)
                while len(los) > 1:
                    los = [los[j] + los[j + 1] for j in range(0, len(los), 2)]
                    his = [his[j] + his[j + 1] for j in range(0, len(his), 2)]
                out.append(lo + los[0])
                out.append(hi + his[0])
            return tuple(out)

        accs = tuple(jnp.zeros((LANES,), jnp.float32) for _ in range(2 * NVREG))
        accs = lax.fori_loop(0, L // UNROLL, row_add, accs)
        for k in range(NVREG):
            out_v[bag, pl.ds(2 * k * LANES, LANES)] = accs[2 * k] * inv
            out_v[bag, pl.ds((2 * k + 1) * LANES, LANES)] = accs[2 * k + 1] * inv

    for s in range(NBUF):
        start_gather(s, s)

    def step(i, _):
        for s in range(NBUF):
            bag = i * NBUF + s
            drain(s)
            consume(s, bag)
            start_gather(s, bag + NBUF)
        return 0

    lax.fori_loop(0, BPW // NBUF - 1, step, 0)

    for s in range(NBUF):
        bag = BPW - NBUF + s
        drain(s)
        consume(s, bag)

    pltpu.sync_copy(out_v, out_hbm.at[pl.ds(wid * BPW, BPW)])


@jax.jit
def _bow(indices, table):
    table_p = _pack_table(table)
    mesh = plsc.VectorSubcoreMesh(core_axis_name="c", subcore_axis_name="s")
    return pl.kernel(
        _bow_body,
        mesh=mesh,
        compiler_params=pltpu.CompilerParams(
            needs_layout_passes=False, use_tc_tiling_on_sc=False),
        out_type=jax.ShapeDtypeStruct((B, D), jnp.float32),
        scratch_types=[
            pltpu.VMEM((BPW, L), jnp.int32),
            pltpu.VMEM((NBUF, L, DW), jnp.int32),
            pltpu.VMEM((BPW, D), jnp.float32),
            pltpu.SemaphoreType.DMA,
            pltpu.SemaphoreType.DMA,
        ],
    )(indices, table_p)


def kernel(indices, table):
    return _bow(indices, table)


# speedup vs baseline: 2.3708x; 2.3708x over previous
import functools

import jax
import jax.numpy as jnp
from jax import lax
from jax.experimental import pallas as pl
from jax.experimental.pallas import tpu as pltpu
from jax.experimental.pallas import tpu_sc as plsc

B = 4096
L = 200
V = 100000
D = 128

NC = 2
NS = 16
LANES = 16
NW = NC * NS
BPW = B // NW
CHUNKS = ((0, 104), (104, 96))
NBUF = 2
DW = D // 2
NVREG = D // 32

CAST_ROWS = 800


def _cast_body(x_ref, o_ref):
    bits = lax.bitcast_convert_type(x_ref[...], jnp.int32)
    r = bits + 0x7FFF + jnp.bitwise_and(
        lax.shift_right_logical(bits, 16), jnp.int32(1))
    lo = lax.shift_right_logical(r[:, :DW], 16)
    hi = jnp.bitwise_and(r[:, DW:], jnp.int32(-65536))
    o_ref[...] = jnp.bitwise_or(hi, lo)


def _pack_table(table):
    return pl.pallas_call(
        _cast_body,
        grid=(V // CAST_ROWS,),
        in_specs=[pl.BlockSpec((CAST_ROWS, D), lambda i: (i, 0))],
        out_specs=pl.BlockSpec((CAST_ROWS, DW), lambda i: (i, 0)),
        out_shape=jax.ShapeDtypeStruct((V, DW), jnp.int32),
    )(table)


def _bow_body(idx_hbm, table_hbm, out_hbm, idx_v, buf_v, out_v, sem0, sem1):
    wid = lax.axis_index("s") * NC + lax.axis_index("c")
    sems = (sem0, sem1)
    inv = jnp.full((LANES,), 1.0 / L, dtype=jnp.float32)

    pltpu.sync_copy(idx_hbm.at[pl.ds(wid * BPW, BPW)], idx_v)

    def start_gather(slot, bag):
        for off, ch in CHUNKS:
            pltpu.make_async_copy(
                table_hbm.at[idx_v.at[bag, pl.ds(off, ch)]],
                buf_v.at[slot, pl.ds(off, ch)],
                sems[slot],
            ).start()

    def drain(slot):
        for off, ch in CHUNKS:
            pltpu.make_async_copy(
                table_hbm.at[idx_v.at[0, pl.ds(off, ch)]],
                buf_v.at[slot, pl.ds(off, ch)],
                sems[slot],
            ).wait()

    def consume(slot, bag):
        UNROLL = 4

        def row_add(i, accs):
            l = i * UNROLL
            out = []
            for k in range(NVREG):
                lo, hi = accs[2 * k], accs[2 * k + 1]
                los, his = [], []
                for u in range(UNROLL):
                    x = buf_v[slot, l + u, pl.ds(k * LANES, LANES)]
                    a, b = plsc.unpack(
                        plsc.bitcast(x, jnp.bfloat16),
                        format=plsc.PackFormat.INTERLEAVED)
                    los.append(a)
                    his.append(b)
                while len(los) > 1:
                    los = [los[j] + los[j + 1] for j in range(0, len(los), 2)]
                    his = [his[j] + his[j + 1] for j in range(0, len(his), 2)]
                out.append(lo + los[0])
                out.append(hi + his[0])
            return tuple(out)

        accs = tuple(jnp.zeros((LANES,), jnp.float32) for _ in range(2 * NVREG))
        accs = lax.fori_loop(0, L // UNROLL, row_add, accs)
        for k in range(NVREG):
            out_v[bag, pl.ds(k * LANES, LANES)] = accs[2 * k] * inv
            out_v[bag, pl.ds(DW + k * LANES, LANES)] = accs[2 * k + 1] * inv

    for s in range(NBUF):
        start_gather(s, s)

    def step(i, _):
        for s in range(NBUF):
            bag = i * NBUF + s
            drain(s)
            consume(s, bag)
            start_gather(s, bag + NBUF)
        return 0

    lax.fori_loop(0, BPW // NBUF - 1, step, 0)

    for s in range(NBUF):
        bag = BPW - NBUF + s
        drain(s)
        consume(s, bag)

    pltpu.sync_copy(out_v, out_hbm.at[pl.ds(wid * BPW, BPW)])


@jax.jit
def _bow(indices, table):
    table_p = _pack_table(table)
    mesh = plsc.VectorSubcoreMesh(core_axis_name="c", subcore_axis_name="s")
    return pl.kernel(
        _bow_body,
        mesh=mesh,
        compiler_params=pltpu.CompilerParams(
            needs_layout_passes=False, use_tc_tiling_on_sc=False),
        out_type=jax.ShapeDtypeStruct((B, D), jnp.float32),
        scratch_types=[
            pltpu.VMEM((BPW, L), jnp.int32),
            pltpu.VMEM((NBUF, L, DW), jnp.int32),
            pltpu.VMEM((BPW, D), jnp.float32),
            pltpu.SemaphoreType.DMA,
            pltpu.SemaphoreType.DMA,
        ],
    )(indices, table_p)


def kernel(indices, table):
    return _bow(indices, table)


# speedup vs baseline: 2.7053x; 1.1411x over previous
import functools

import jax
import jax.numpy as jnp
from jax import lax
from jax.experimental import pallas as pl
from jax.experimental.pallas import tpu as pltpu
from jax.experimental.pallas import tpu_sc as plsc

B = 4096
L = 200
V = 100000
D = 128

NC = 2
NS = 16
LANES = 16
NW = NC * NS
BPW = B // NW
CHUNKS = ((0, 104), (104, 96))
NBUF = 2
DW = D // 2
NVREG = D // 32

CAST_ROWS = 2000


def _cast_body(x_ref, o_ref):
    bits = lax.bitcast_convert_type(x_ref[...], jnp.int32)
    r = bits + 0x7FFF + jnp.bitwise_and(
        lax.shift_right_logical(bits, 16), jnp.int32(1))
    lo = lax.shift_right_logical(r[:, :DW], 16)
    hi = jnp.bitwise_and(r[:, DW:], jnp.int32(-65536))
    o_ref[...] = jnp.bitwise_or(hi, lo)


def _pack_table(table):
    return pl.pallas_call(
        _cast_body,
        grid=(V // CAST_ROWS,),
        in_specs=[pl.BlockSpec((CAST_ROWS, D), lambda i: (i, 0))],
        out_specs=pl.BlockSpec((CAST_ROWS, DW), lambda i: (i, 0)),
        out_shape=jax.ShapeDtypeStruct((V, DW), jnp.int32),
    )(table)


def _bow_body(idx_hbm, table_hbm, out_hbm, idx_v, buf_v, out_v, sem0, sem1):
    wid = lax.axis_index("s") * NC + lax.axis_index("c")
    sems = (sem0, sem1)
    inv = jnp.full((LANES,), 1.0 / L, dtype=jnp.float32)

    pltpu.sync_copy(idx_hbm.at[pl.ds(wid * BPW, BPW)], idx_v)

    def start_gather(slot, bag):
        for off, ch in CHUNKS:
            pltpu.make_async_copy(
                table_hbm.at[idx_v.at[bag, pl.ds(off, ch)]],
                buf_v.at[slot, pl.ds(off, ch)],
                sems[slot],
            ).start()

    def drain(slot):
        for off, ch in CHUNKS:
            pltpu.make_async_copy(
                table_hbm.at[idx_v.at[0, pl.ds(off, ch)]],
                buf_v.at[slot, pl.ds(off, ch)],
                sems[slot],
            ).wait()

    def consume(slot, bag):
        UNROLL = 4

        def row_add(i, accs):
            l = i * UNROLL
            out = []
            for k in range(NVREG):
                lo, hi = accs[2 * k], accs[2 * k + 1]
                los, his = [], []
                for u in range(UNROLL):
                    x = buf_v[slot, l + u, pl.ds(k * LANES, LANES)]
                    a, b = plsc.unpack(
                        plsc.bitcast(x, jnp.bfloat16),
                        format=plsc.PackFormat.INTERLEAVED)
                    los.append(a)
                    his.append(b)
                while len(los) > 1:
                    los = [los[j] + los[j + 1] for j in range(0, len(los), 2)]
                    his = [his[j] + his[j + 1] for j in range(0, len(his), 2)]
                out.append(lo + los[0])
                out.append(hi + his[0])
            return tuple(out)

        accs = tuple(jnp.zeros((LANES,), jnp.float32) for _ in range(2 * NVREG))
        accs = lax.fori_loop(0, L // UNROLL, row_add, accs)
        for k in range(NVREG):
            out_v[bag, pl.ds(k * LANES, LANES)] = accs[2 * k] * inv
            out_v[bag, pl.ds(DW + k * LANES, LANES)] = accs[2 * k + 1] * inv

    for s in range(NBUF):
        start_gather(s, s)

    def step(i, _):
        for s in range(NBUF):
            bag = i * NBUF + s
            drain(s)
            consume(s, bag)
            start_gather(s, bag + NBUF)
        return 0

    lax.fori_loop(0, BPW // NBUF - 1, step, 0)

    for s in range(NBUF):
        bag = BPW - NBUF + s
        drain(s)
        consume(s, bag)

    pltpu.sync_copy(out_v, out_hbm.at[pl.ds(wid * BPW, BPW)])


@jax.jit
def _bow(indices, table):
    table_p = _pack_table(table)
    mesh = plsc.VectorSubcoreMesh(core_axis_name="c", subcore_axis_name="s")
    return pl.kernel(
        _bow_body,
        mesh=mesh,
        compiler_params=pltpu.CompilerParams(
            needs_layout_passes=False, use_tc_tiling_on_sc=False),
        out_type=jax.ShapeDtypeStruct((B, D), jnp.float32),
        scratch_types=[
            pltpu.VMEM((BPW, L), jnp.int32),
            pltpu.VMEM((NBUF, L, DW), jnp.int32),
            pltpu.VMEM((BPW, D), jnp.float32),
            pltpu.SemaphoreType.DMA,
            pltpu.SemaphoreType.DMA,
        ],
    )(indices, table_p)


def kernel(indices, table):
    return _bow(indices, table)


# speedup vs baseline: 3.2293x; 1.1937x over previous
import functools

import jax
import jax.numpy as jnp
from jax import lax
from jax.experimental import pallas as pl
from jax.experimental.pallas import tpu as pltpu
from jax.experimental.pallas import tpu_sc as plsc

B = 4096
L = 200
V = 100000
D = 128

NC = 2
NS = 16
LANES = 16
NW = NC * NS
BPW = B // NW
CHUNKS = ((0, 104), (104, 96))
NBUF = 2
NVREG = D // LANES


def _bow_body(idx_hbm, table_hbm, out_hbm, idx_v, buf_v, out_v, sem0, sem1):
    wid = lax.axis_index("s") * NC + lax.axis_index("c")
    sems = (sem0, sem1)
    inv = jnp.full((LANES,), 1.0 / L, dtype=jnp.float32)

    pltpu.sync_copy(idx_hbm.at[pl.ds(wid * BPW, BPW)], idx_v)

    def start_gather(slot, bag):
        for off, ch in CHUNKS:
            pltpu.make_async_copy(
                table_hbm.at[idx_v.at[bag, pl.ds(off, ch)]],
                buf_v.at[slot, pl.ds(off, ch)],
                sems[slot],
            ).start()

    def drain(slot):
        for off, ch in CHUNKS:
            pltpu.make_async_copy(
                table_hbm.at[idx_v.at[0, pl.ds(off, ch)]],
                buf_v.at[slot, pl.ds(off, ch)],
                sems[slot],
            ).wait()

    def consume(slot, bag):
        UNROLL = 2

        def row_add(i, accs):
            l = i * UNROLL
            out = []
            for k in range(NVREG):
                a = accs[k]
                parts = [
                    buf_v[slot, l + u, pl.ds(k * LANES, LANES)]
                    for u in range(UNROLL)
                ]
                while len(parts) > 1:
                    parts = [
                        parts[j] + parts[j + 1]
                        for j in range(0, len(parts), 2)
                    ]
                out.append(a + parts[0])
            return tuple(out)

        accs = tuple(jnp.zeros((LANES,), jnp.float32) for _ in range(NVREG))
        accs = lax.fori_loop(0, L // UNROLL, row_add, accs)
        for k in range(NVREG):
            out_v[bag, pl.ds(k * LANES, LANES)] = accs[k] * inv

    for s in range(NBUF):
        start_gather(s, s)

    def step(i, _):
        for s in range(NBUF):
            bag = i * NBUF + s
            drain(s)
            consume(s, bag)
            start_gather(s, bag + NBUF)
        return 0

    lax.fori_loop(0, BPW // NBUF - 1, step, 0)

    for s in range(NBUF):
        bag = BPW - NBUF + s
        drain(s)
        consume(s, bag)

    pltpu.sync_copy(out_v, out_hbm.at[pl.ds(wid * BPW, BPW)])


@jax.jit
def _bow(indices, table):
    mesh = plsc.VectorSubcoreMesh(core_axis_name="c", subcore_axis_name="s")
    return pl.kernel(
        _bow_body,
        mesh=mesh,
        compiler_params=pltpu.CompilerParams(
            needs_layout_passes=False, use_tc_tiling_on_sc=False),
        out_type=jax.ShapeDtypeStruct((B, D), jnp.float32),
        scratch_types=[
            pltpu.VMEM((BPW, L), jnp.int32),
            pltpu.VMEM((NBUF, L, D), jnp.float32),
            pltpu.VMEM((BPW, D), jnp.float32),
            pltpu.SemaphoreType.DMA,
            pltpu.SemaphoreType.DMA,
        ],
    )(indices, table)


def kernel(indices, table):
    return _bow(indices, table)


# speedup vs baseline: 3.9136x; 1.2119x over previous
import functools

import jax
import jax.numpy as jnp
from jax import lax
from jax.experimental import pallas as pl
from jax.experimental.pallas import tpu as pltpu
from jax.experimental.pallas import tpu_sc as plsc

B = 4096
L = 200
V = 100000
D = 128

NC = 2
NS = 16
LANES = 16
NW = NC * NS
BPW = B // NW
CHUNKS = ((0, 104), (104, 96))
CHMAX = 104
NBUF = 4
NVREG = D // LANES


def _bow_body(idx_hbm, table_hbm, out_hbm, idx_v, buf_v, out_v,
              sem0, sem1, sem2, sem3):
    wid = lax.axis_index("s") * NC + lax.axis_index("c")
    sems = (sem0, sem1, sem2, sem3)
    inv = jnp.full((LANES,), 1.0 / L, dtype=jnp.float32)

    pltpu.sync_copy(idx_hbm.at[pl.ds(wid * BPW, BPW)], idx_v)

    def start_gather(slot, bag, ci):
        off, ch = CHUNKS[ci]
        pltpu.make_async_copy(
            table_hbm.at[idx_v.at[bag, pl.ds(off, ch)]],
            buf_v.at[slot, pl.ds(0, ch)],
            sems[slot],
        ).start()

    def drain(slot, ci):
        off, ch = CHUNKS[ci]
        pltpu.make_async_copy(
            table_hbm.at[idx_v.at[0, pl.ds(off, ch)]],
            buf_v.at[slot, pl.ds(0, ch)],
            sems[slot],
        ).wait()

    UNROLL = 2

    def consume(slot, ci, accs):
        _, ch = CHUNKS[ci]

        def row_add(i, accs):
            l = i * UNROLL
            out = []
            for k in range(NVREG):
                a = accs[k]
                parts = [
                    buf_v[slot, l + u, pl.ds(k * LANES, LANES)]
                    for u in range(UNROLL)
                ]
                while len(parts) > 1:
                    parts = [
                        parts[j] + parts[j + 1]
                        for j in range(0, len(parts), 2)
                    ]
                out.append(a + parts[0])
            return tuple(out)

        return lax.fori_loop(0, ch // UNROLL, row_add, accs)

    def store(bag, accs):
        for k in range(NVREG):
            out_v[bag, pl.ds(k * LANES, LANES)] = accs[k] * inv

    for bagoff in range(2):
        for ci in range(2):
            start_gather(2 * bagoff + ci, bagoff, ci)

    def step(i, _):
        for bagoff in range(2):
            bag = 2 * i + bagoff
            accs = tuple(
                jnp.zeros((LANES,), jnp.float32) for _ in range(NVREG))
            for ci in range(2):
                s = 2 * bagoff + ci
                drain(s, ci)
                accs = consume(s, ci, accs)
                start_gather(s, bag + 2, ci)
            store(bag, accs)
        return 0

    lax.fori_loop(0, BPW // 2 - 1, step, 0)

    for bagoff in range(2):
        bag = BPW - 2 + bagoff
        accs = tuple(jnp.zeros((LANES,), jnp.float32) for _ in range(NVREG))
        for ci in range(2):
            s = 2 * bagoff + ci
            drain(s, ci)
            accs = consume(s, ci, accs)
        store(bag, accs)

    pltpu.sync_copy(out_v, out_hbm.at[pl.ds(wid * BPW, BPW)])


@jax.jit
def _bow(indices, table):
    mesh = plsc.VectorSubcoreMesh(core_axis_name="c", subcore_axis_name="s")
    return pl.kernel(
        _bow_body,
        mesh=mesh,
        compiler_params=pltpu.CompilerParams(
            needs_layout_passes=False, use_tc_tiling_on_sc=False),
        out_type=jax.ShapeDtypeStruct((B, D), jnp.float32),
        scratch_types=[
            pltpu.VMEM((BPW, L), jnp.int32),
            pltpu.VMEM((NBUF, CHMAX, D), jnp.float32),
            pltpu.VMEM((BPW, D), jnp.float32),
            pltpu.SemaphoreType.DMA,
            pltpu.SemaphoreType.DMA,
            pltpu.SemaphoreType.DMA,
            pltpu.SemaphoreType.DMA,
        ],
    )(indices, table)


def kernel(indices, table):
    return _bow(indices, table)


# speedup vs baseline: 3.9490x; 1.0090x over previous
import functools

import jax
import jax.numpy as jnp
from jax import lax
from jax.experimental import pallas as pl
from jax.experimental.pallas import tpu as pltpu
from jax.experimental.pallas import tpu_sc as plsc

B = 4096
L = 200
V = 100000
D = 128

NC = 2
NS = 16
LANES = 16
NW = NC * NS
BPW = B // NW
CHUNKS = ((0, 104), (104, 96))
CHMAX = 104
NBUF = 6
NVREG = D // LANES


def _bow_body(idx_hbm, table_hbm, out_hbm, idx_v, buf_v, out_v,
              sem0, sem1, sem2, sem3, sem4, sem5):
    wid = lax.axis_index("s") * NC + lax.axis_index("c")
    sems = (sem0, sem1, sem2, sem3, sem4, sem5)
    inv = jnp.full((LANES,), 1.0 / L, dtype=jnp.float32)

    pltpu.sync_copy(idx_hbm.at[pl.ds(wid * BPW, BPW)], idx_v)

    def start_gather(slot, bag, ci):
        off, ch = CHUNKS[ci]
        pltpu.make_async_copy(
            table_hbm.at[idx_v.at[bag, pl.ds(off, ch)]],
            buf_v.at[slot, pl.ds(0, ch)],
            sems[slot],
        ).start()

    def drain(slot, ci):
        off, ch = CHUNKS[ci]
        pltpu.make_async_copy(
            table_hbm.at[idx_v.at[0, pl.ds(off, ch)]],
            buf_v.at[slot, pl.ds(0, ch)],
            sems[slot],
        ).wait()

    UNROLL = 2

    def consume(slot, ci, accs):
        _, ch = CHUNKS[ci]

        def row_add(i, accs):
            l = i * UNROLL
            out = []
            for k in range(NVREG):
                a = accs[k]
                parts = [
                    buf_v[slot, l + u, pl.ds(k * LANES, LANES)]
                    for u in range(UNROLL)
                ]
                while len(parts) > 1:
                    parts = [
                        parts[j] + parts[j + 1]
                        for j in range(0, len(parts), 2)
                    ]
                out.append(a + parts[0])
            return tuple(out)

        return lax.fori_loop(0, ch // UNROLL, row_add, accs)

    def store(bag, accs):
        for k in range(NVREG):
            out_v[bag, pl.ds(k * LANES, LANES)] = accs[k] * inv

    GRP = NBUF // 2
    NSTEPS = 41

    for bagoff in range(GRP):
        for ci in range(2):
            start_gather(2 * bagoff + ci, bagoff, ci)

    def step(i, _):
        for bagoff in range(GRP):
            bag = GRP * i + bagoff
            accs = tuple(
                jnp.zeros((LANES,), jnp.float32) for _ in range(NVREG))
            for ci in range(2):
                s = 2 * bagoff + ci
                drain(s, ci)
                accs = consume(s, ci, accs)
                start_gather(s, bag + GRP, ci)
            store(bag, accs)
        return 0

    lax.fori_loop(0, NSTEPS, step, 0)

    for bag, rbag in ((123, 126), (124, 127), (125, None), (126, None),
                      (127, None)):
        s0 = (2 * bag) % NBUF
        accs = tuple(jnp.zeros((LANES,), jnp.float32) for _ in range(NVREG))
        for ci in range(2):
            drain(s0 + ci, ci)
            accs = consume(s0 + ci, ci, accs)
            if rbag is not None:
                start_gather(s0 + ci, rbag, ci)
        store(bag, accs)

    pltpu.sync_copy(out_v, out_hbm.at[pl.ds(wid * BPW, BPW)])


@jax.jit
def _bow(indices, table):
    mesh = plsc.VectorSubcoreMesh(core_axis_name="c", subcore_axis_name="s")
    return pl.kernel(
        _bow_body,
        mesh=mesh,
        compiler_params=pltpu.CompilerParams(
            needs_layout_passes=False, use_tc_tiling_on_sc=False),
        out_type=jax.ShapeDtypeStruct((B, D), jnp.float32),
        scratch_types=[
            pltpu.VMEM((BPW, L), jnp.int32),
            pltpu.VMEM((NBUF, CHMAX, D), jnp.float32),
            pltpu.VMEM((BPW, D), jnp.float32),
            pltpu.SemaphoreType.DMA,
            pltpu.SemaphoreType.DMA,
            pltpu.SemaphoreType.DMA,
            pltpu.SemaphoreType.DMA,
            pltpu.SemaphoreType.DMA,
            pltpu.SemaphoreType.DMA,
        ],
    )(indices, table)


def kernel(indices, table):
    return _bow(indices, table)
